# baseline (speedup 1.0000x reference)
import jax
import jax.numpy as jnp
from jax import lax
from jax.experimental import pallas as pl
from jax.experimental.pallas import tpu as pltpu
from jax.experimental.pallas import tpu_sc as plsc

_VOCAB = 100000
_DIM = 64
_NC = 2
_NS = 16
_NW = _NC * _NS

_B = 4096
_J = 200
_IBLK = _B // _NW
_NCHUNK = _J
_NBUF = 4
_NTB = 2
_A = 2


def _gather_body(idx_hbm, tab_hbm, out_hbm, *scratch):
    idxs = scratch[0:_NBUF]
    rows = scratch[_NBUF:2 * _NBUF]
    trs = scratch[2 * _NBUF:2 * _NBUF + _NTB]
    si = scratch[2 * _NBUF + _NTB:3 * _NBUF + _NTB]
    sg = scratch[3 * _NBUF + _NTB:4 * _NBUF + _NTB]
    sw = scratch[4 * _NBUF + _NTB:4 * _NBUF + 2 * _NTB]

    wid = lax.axis_index("s") * _NC + lax.axis_index("c")
    i0 = wid * _IBLK

    def fire_idx(c, b):
        pltpu.async_copy(idx_hbm.at[c, pl.ds(i0, _IBLK)], idxs[b], si[b])

    def wait_idx(b):
        pltpu.make_async_copy(idx_hbm.at[0, pl.ds(i0, _IBLK)],
                              idxs[b], si[b]).wait()

    def fire_gather(b):
        pltpu.async_copy(tab_hbm.at[idxs[b]], rows[b], sg[b])

    def wait_gather(b):
        pltpu.make_async_copy(tab_hbm.at[idxs[b]], rows[b], sg[b]).wait()

    def fire_wb(c, t):
        pltpu.async_copy(trs[t], out_hbm.at[c, :, wid], sw[t])

    def wait_wb(t):
        pltpu.make_async_copy(trs[t], out_hbm.at[0, :, wid], sw[t]).wait()

    lanes = [jnp.arange(16 * g, 16 * (g + 1), dtype=jnp.int32)
             for g in range(_IBLK // 16)]

    def transpose_chunk(b, t):
        def dt_step(dt, carry):
            for dr in range(8):
                d = jnp.full((16,), dt * 8 + dr, dtype=jnp.int32)
                for g in range(_IBLK // 16):
                    v = plsc.load_gather(rows[b], [lanes[g], d])
                    trs[t][dt, dr, pl.ds(16 * g, 16)] = v
            return carry

        lax.fori_loop(0, 8, dt_step, 0)

    for c in range(_A + 1):
        fire_idx(c, c % _NBUF)
    for c in range(_A):
        wait_idx(c % _NBUF)
        fire_gather(c % _NBUF)

    _STEP = _NBUF * _NTB // (2 if _NBUF % 2 == 0 and _NTB == 2 else 1)

    def group(g, carry):
        for s in range(_STEP):
            k = g * _STEP + s
            b = s % _NBUF
            t = s % _NTB
            ba = (s + _A) % _NBUF
            bn = (s + _A + 1) % _NBUF

            @pl.when(k + _A < _NCHUNK)
            def _():
                wait_idx(ba)
                fire_gather(ba)

            wait_gather(b)

            @pl.when(k >= _NTB)
            def _():
                wait_wb(t)

            transpose_chunk(b, t)
            fire_wb(k, t)

            @pl.when(k + _A + 1 < _NCHUNK)
            def _():
                fire_idx(k + _A + 1, bn)
        return carry

    lax.fori_loop(0, _NCHUNK // _STEP, group, 0)

    for t in range(_NTB):
        wait_wb(t)


_mesh = plsc.VectorSubcoreMesh(core_axis_name="c", subcore_axis_name="s")


def kernel(tensor, table_fix, table_v):
    idx = jnp.swapaxes(tensor, 0, 1).astype(jnp.int32)
    tab = table_fix.at[_VOCAB - 3:].add(table_v)
    call = pl.kernel(
        _gather_body,
        out_type=jax.ShapeDtypeStruct((_J, 8, _NW, 8, 128), jnp.float32),
        mesh=_mesh,
        scratch_types=(
            [pltpu.VMEM((_IBLK,), jnp.int32) for _ in range(_NBUF)]
            + [pltpu.VMEM((_IBLK, _DIM), jnp.float32) for _ in range(_NBUF)]
            + [pltpu.VMEM((8, 8, 128), jnp.float32) for _ in range(_NTB)]
            + [pltpu.SemaphoreType.DMA for _ in range(2 * _NBUF + _NTB)]
        ),
        compiler_params=pltpu.CompilerParams(use_tc_tiling_on_sc=False,
                                             needs_layout_passes=False),
    )
    out5 = call(idx, tab)
    out = jnp.transpose(out5, (0, 2, 4, 1, 3)).reshape(_J, _B, _DIM)
    return out


# speedup vs baseline: 1.2650x; 1.2650x over previous
import jax
import jax.numpy as jnp
from jax import lax
from jax.experimental import pallas as pl
from jax.experimental.pallas import tpu as pltpu
from jax.experimental.pallas import tpu_sc as plsc

_VOCAB = 100000
_DIM = 64
_NC = 2
_NS = 16
_NW = _NC * _NS

_B = 4096
_J = 200
_IBLK = _B // _NW
_NCHUNK = _J
_NBUF = 4
_NTB = 2
_A = 2


def _gather_body(idx_hbm, tab_hbm, out_hbm, *scratch):
    idxs = scratch[0:_NBUF]
    rows = scratch[_NBUF:2 * _NBUF]
    trs = scratch[2 * _NBUF:2 * _NBUF + _NTB]
    si = scratch[2 * _NBUF + _NTB:3 * _NBUF + _NTB]
    sg = scratch[3 * _NBUF + _NTB:4 * _NBUF + _NTB]
    sw = scratch[4 * _NBUF + _NTB:4 * _NBUF + 2 * _NTB]

    wid = lax.axis_index("s") * _NC + lax.axis_index("c")
    i0 = wid * _IBLK

    def fire_idx(c, b):
        pltpu.async_copy(idx_hbm.at[c, pl.ds(i0, _IBLK)], idxs[b], si[b])

    def wait_idx(b):
        pltpu.make_async_copy(idx_hbm.at[0, pl.ds(i0, _IBLK)],
                              idxs[b], si[b]).wait()

    def fire_gather(b):
        pltpu.async_copy(tab_hbm.at[idxs[b]], rows[b], sg[b])

    def wait_gather(b):
        pltpu.make_async_copy(tab_hbm.at[idxs[b]], rows[b], sg[b]).wait()

    def fire_wb(c, t):
        pltpu.async_copy(trs[t], out_hbm.at[c, :, wid], sw[t])

    def wait_wb(t):
        pltpu.make_async_copy(trs[t], out_hbm.at[0, :, wid], sw[t]).wait()

    lanes = [jnp.arange(16 * g, 16 * (g + 1), dtype=jnp.int32)
             for g in range(_IBLK // 16)]

    def transpose_chunk(b, t):
        @plsc.parallel_loop(0, 8, unroll=2)
        def _(dt):
            for dr in range(8):
                d = jnp.full((16,), dt * 8 + dr, dtype=jnp.int32)
                vs = [plsc.load_gather(rows[b], [lanes[g], d])
                      for g in range(_IBLK // 16)]
                for g in range(_IBLK // 16):
                    trs[t][dt, dr, pl.ds(16 * g, 16)] = vs[g]

    for c in range(_A + 1):
        fire_idx(c, c % _NBUF)
    for c in range(_A):
        wait_idx(c % _NBUF)
        fire_gather(c % _NBUF)

    _STEP = 4

    def group(g, carry):
        for s in range(_STEP):
            k = g * _STEP + s
            b = s % _NBUF
            t = s % _NTB
            ba = (s + _A) % _NBUF
            bn = (s + _A + 1) % _NBUF

            @pl.when(k + _A < _NCHUNK)
            def _():
                wait_idx(ba)
                fire_gather(ba)

            wait_gather(b)

            @pl.when(k >= _NTB)
            def _():
                wait_wb(t)

            transpose_chunk(b, t)
            fire_wb(k, t)

            @pl.when(k + _A + 1 < _NCHUNK)
            def _():
                fire_idx(k + _A + 1, bn)
        return carry

    lax.fori_loop(0, _NCHUNK // _STEP, group, 0)

    for t in range(_NTB):
        wait_wb(t)


_mesh = plsc.VectorSubcoreMesh(core_axis_name="c", subcore_axis_name="s")


def kernel(tensor, table_fix, table_v):
    idx = jnp.swapaxes(tensor, 0, 1).astype(jnp.int32)
    tab = table_fix.at[_VOCAB - 3:].add(table_v)
    call = pl.kernel(
        _gather_body,
        out_type=jax.ShapeDtypeStruct((_J, 8, _NW, 8, 128), jnp.float32),
        mesh=_mesh,
        scratch_types=(
            [pltpu.VMEM((_IBLK,), jnp.int32) for _ in range(_NBUF)]
            + [pltpu.VMEM((_IBLK, _DIM), jnp.float32) for _ in range(_NBUF)]
            + [pltpu.VMEM((8, 8, 128), jnp.float32) for _ in range(_NTB)]
            + [pltpu.SemaphoreType.DMA for _ in range(2 * _NBUF + _NTB)]
        ),
        compiler_params=pltpu.CompilerParams(use_tc_tiling_on_sc=False,
                                             needs_layout_passes=False),
    )
    out5 = call(idx, tab)
    out = jnp.transpose(out5, (0, 2, 4, 1, 3)).reshape(_J, _B, _DIM)
    return out


# speedup vs baseline: 1.8218x; 1.4402x over previous
import jax
import jax.numpy as jnp
from jax import lax
from jax.experimental import pallas as pl
from jax.experimental.pallas import tpu as pltpu
from jax.experimental.pallas import tpu_sc as plsc

_VOCAB = 100000
_DIM = 64
_NC = 2
_NS = 16
_NW = _NC * _NS

_B = 4096
_J = 200
_IBLK = _B // _NW
_NCHUNK = _J
_NBUF = 4
_NTB = 2
_A = 2


def _gather_body(idx_hbm, tab_hbm, out_hbm, *scratch):
    idxs = scratch[0:_NBUF]
    rows = scratch[_NBUF:2 * _NBUF]
    trs = scratch[2 * _NBUF:2 * _NBUF + _NTB]
    si = scratch[2 * _NBUF + _NTB:3 * _NBUF + _NTB]
    sg = scratch[3 * _NBUF + _NTB:4 * _NBUF + _NTB]
    sw = scratch[4 * _NBUF + _NTB:4 * _NBUF + 2 * _NTB]

    wid = lax.axis_index("s") * _NC + lax.axis_index("c")
    i0 = wid * _IBLK

    def fire_idx(c, b):
        pltpu.async_copy(idx_hbm.at[c, pl.ds(i0, _IBLK)], idxs[b], si[b])

    def wait_idx(b):
        pltpu.make_async_copy(idx_hbm.at[0, pl.ds(i0, _IBLK)],
                              idxs[b], si[b]).wait()

    def fire_gather(b):
        pltpu.async_copy(tab_hbm.at[idxs[b]], rows[b], sg[b])

    def wait_gather(b):
        pltpu.make_async_copy(tab_hbm.at[idxs[b]], rows[b], sg[b]).wait()

    def fire_wb(c, t):
        for dt in range(8):
            pltpu.async_copy(trs[t].at[pl.ds(dt * 1024, 1024)],
                             out_hbm.at[c, dt, wid], sw[t])

    def wait_wb(t):
        for dt in range(8):
            pltpu.make_async_copy(trs[t].at[pl.ds(0, 1024)],
                                  out_hbm.at[0, 0, wid], sw[t]).wait()

    iota = jnp.arange(16, dtype=jnp.int32)
    rot = [(iota + s) % 16 for s in range(16)]
    ivec = [16 * g + iota for g in range(_IBLK // 16)]
    mvec = [((iota + s) % 16) * 128 + iota for s in range(16)]

    def transpose_chunk(b, t):
        @plsc.parallel_loop(0, 4, unroll=1)
        def _(dblk):
            d0 = dblk * 16
            d0v = jnp.full((16,), d0, dtype=jnp.int32)
            for s in range(16):
                dv = d0v + rot[s]
                for g in range(_IBLK // 16):
                    v = plsc.load_gather(rows[b], [ivec[g], dv])
                    sidx = mvec[s] + (d0 * 128 + 16 * g)
                    plsc.store_scatter(trs[t], [sidx], v)

    for c in range(_A + 1):
        fire_idx(c, c % _NBUF)
    for c in range(_A):
        wait_idx(c % _NBUF)
        fire_gather(c % _NBUF)

    _STEP = 4

    def group(g, carry):
        for s in range(_STEP):
            k = g * _STEP + s
            b = s % _NBUF
            t = s % _NTB
            ba = (s + _A) % _NBUF
            bn = (s + _A + 1) % _NBUF

            @pl.when(k + _A < _NCHUNK)
            def _():
                wait_idx(ba)
                fire_gather(ba)

            wait_gather(b)

            @pl.when(k >= _NTB)
            def _():
                wait_wb(t)

            transpose_chunk(b, t)
            fire_wb(k, t)

            @pl.when(k + _A + 1 < _NCHUNK)
            def _():
                fire_idx(k + _A + 1, bn)
        return carry

    lax.fori_loop(0, _NCHUNK // _STEP, group, 0)

    for t in range(_NTB):
        wait_wb(t)


_mesh = plsc.VectorSubcoreMesh(core_axis_name="c", subcore_axis_name="s")


def kernel(tensor, table_fix, table_v):
    idx = jnp.swapaxes(tensor, 0, 1).astype(jnp.int32)
    tab = table_fix.at[_VOCAB - 3:].add(table_v)
    call = pl.kernel(
        _gather_body,
        out_type=jax.ShapeDtypeStruct((_J, 8, _NW, 1024), jnp.float32),
        mesh=_mesh,
        scratch_types=(
            [pltpu.VMEM((_IBLK,), jnp.int32) for _ in range(_NBUF)]
            + [pltpu.VMEM((_IBLK, _DIM), jnp.float32) for _ in range(_NBUF)]
            + [pltpu.VMEM((8 * 1024,), jnp.float32) for _ in range(_NTB)]
            + [pltpu.SemaphoreType.DMA for _ in range(2 * _NBUF + _NTB)]
        ),
        compiler_params=pltpu.CompilerParams(use_tc_tiling_on_sc=False,
                                             needs_layout_passes=False),
    )
    out4 = call(idx, tab)
    out5 = out4.reshape(_J, 8, _NW, 8, 128)
    out = jnp.transpose(out5, (0, 2, 4, 1, 3)).reshape(_J, _B, _DIM)
    return out


# speedup vs baseline: 2.9744x; 1.6327x over previous
import jax
import jax.numpy as jnp
from jax import lax
from jax.experimental import pallas as pl
from jax.experimental.pallas import tpu as pltpu
from jax.experimental.pallas import tpu_sc as plsc

_VOCAB = 100000
_DIM = 64
_NC = 2
_NS = 16
_NW = _NC * _NS

_B = 4096
_J = 200
_IBLK = _B // _NW
_NCHUNK = _J
_NBUF = 4
_NTB = 2
_A = 2


def _gather_body(idx_hbm, tab_hbm, out_hbm, *scratch):
    idxs = scratch[0:_NBUF]
    rows = scratch[_NBUF:2 * _NBUF]
    trs = scratch[2 * _NBUF:2 * _NBUF + _NTB]
    si = scratch[2 * _NBUF + _NTB:3 * _NBUF + _NTB]
    sg = scratch[3 * _NBUF + _NTB:4 * _NBUF + _NTB]
    sw = scratch[4 * _NBUF + _NTB:4 * _NBUF + 2 * _NTB]

    wid = lax.axis_index("s") * _NC + lax.axis_index("c")
    i0 = wid * _IBLK

    def fire_idx(c, b):
        pltpu.async_copy(idx_hbm.at[c, pl.ds(i0, _IBLK)], idxs[b], si[b])

    def wait_idx(b):
        pltpu.make_async_copy(idx_hbm.at[0, pl.ds(i0, _IBLK)],
                              idxs[b], si[b]).wait()

    def fire_gather(b):
        pltpu.async_copy(tab_hbm.at[idxs[b]], rows[b], sg[b])

    def wait_gather(b):
        pltpu.make_async_copy(tab_hbm.at[idxs[b]], rows[b], sg[b]).wait()

    def fire_wb(c, t):
        for dt in range(8):
            pltpu.async_copy(trs[t].at[pl.ds(dt * 1024, 1024)],
                             out_hbm.at[c, dt, wid], sw[t])

    def wait_wb(t):
        for dt in range(8):
            pltpu.make_async_copy(trs[t].at[pl.ds(0, 1024)],
                                  out_hbm.at[0, 0, wid], sw[t]).wait()

    iota = jnp.arange(16, dtype=jnp.int32)
    rot = [(iota + s) % 16 for s in range(16)]
    ivec = [16 * g + iota for g in range(_IBLK // 16)]
    mvec = [((iota + s) % 16) * 128 + iota for s in range(16)]

    def transpose_chunk(b, t):
        @plsc.parallel_loop(0, 4, unroll=1)
        def _(dblk):
            d0 = dblk * 16
            d0v = jnp.full((16,), d0, dtype=jnp.int32)
            for s in range(16):
                dv = d0v + rot[s]
                vs = [plsc.load_gather(rows[b], [ivec[g], dv])
                      for g in range(_IBLK // 16)]
                for g in range(_IBLK // 16):
                    sidx = mvec[s] + (d0 * 128 + 16 * g)
                    plsc.store_scatter(trs[t], [sidx], vs[g])

    for c in range(_A + 1):
        fire_idx(c, c % _NBUF)
    for c in range(_A):
        wait_idx(c % _NBUF)
        fire_gather(c % _NBUF)

    _STEP = 4

    def group(g, carry):
        for s in range(_STEP):
            k = g * _STEP + s
            b = s % _NBUF
            t = s % _NTB
            ba = (s + _A) % _NBUF
            bn = (s + _A + 1) % _NBUF

            @pl.when(k + _A < _NCHUNK)
            def _():
                wait_idx(ba)
                fire_gather(ba)

            wait_gather(b)

            @pl.when(k >= _NTB)
            def _():
                wait_wb(t)

            transpose_chunk(b, t)
            fire_wb(k, t)

            @pl.when(k + _A + 1 < _NCHUNK)
            def _():
                fire_idx(k + _A + 1, bn)
        return carry

    lax.fori_loop(0, _NCHUNK // _STEP, group, 0)

    for t in range(_NTB):
        wait_wb(t)


_mesh = plsc.VectorSubcoreMesh(core_axis_name="c", subcore_axis_name="s")


def kernel(tensor, table_fix, table_v):
    idx = jnp.swapaxes(tensor, 0, 1).astype(jnp.int32)
    tab = table_fix.at[_VOCAB - 3:].add(table_v)
    call = pl.kernel(
        _gather_body,
        out_type=jax.ShapeDtypeStruct((_J, 8, _NW, 1024), jnp.float32),
        mesh=_mesh,
        scratch_types=(
            [pltpu.VMEM((_IBLK,), jnp.int32) for _ in range(_NBUF)]
            + [pltpu.VMEM((_IBLK, _DIM), jnp.float32) for _ in range(_NBUF)]
            + [pltpu.VMEM((8 * 1024,), jnp.float32) for _ in range(_NTB)]
            + [pltpu.SemaphoreType.DMA for _ in range(2 * _NBUF + _NTB)]
        ),
        compiler_params=pltpu.CompilerParams(use_tc_tiling_on_sc=False,
                                             needs_layout_passes=False),
    )
    out4 = call(idx, tab)
    out5 = out4.reshape(_J, 8, _NW, 8, 128)
    out = jnp.transpose(out5, (0, 2, 4, 1, 3)).reshape(_J, _B, _DIM)
    return out


# speedup vs baseline: 4.1392x; 1.3916x over previous
import jax
import jax.numpy as jnp
from jax import lax
from jax.experimental import pallas as pl
from jax.experimental.pallas import tpu as pltpu
from jax.experimental.pallas import tpu_sc as plsc

_VOCAB = 100000
_DIM = 64
_NC = 2
_NS = 16
_NW = _NC * _NS

_B = 4096
_J = 200
_IBLK = _B // _NW
_NCHUNK = _J
_NBUF = 4
_NTB = 2
_A = 2


def _gather_body(idx_hbm, tab_hbm, out_hbm, *scratch):
    idxs = scratch[0:_NBUF]
    rows = scratch[_NBUF:2 * _NBUF]
    trs = scratch[2 * _NBUF:2 * _NBUF + _NTB]
    si = scratch[2 * _NBUF + _NTB:3 * _NBUF + _NTB]
    sg = scratch[3 * _NBUF + _NTB:4 * _NBUF + _NTB]
    sw = scratch[4 * _NBUF + _NTB:4 * _NBUF + 2 * _NTB]

    wid = lax.axis_index("s") * _NC + lax.axis_index("c")
    i0 = wid * _IBLK

    def fire_idx(c, b):
        pltpu.async_copy(idx_hbm.at[c, pl.ds(i0, _IBLK)], idxs[b], si[b])

    def wait_idx(b):
        pltpu.make_async_copy(idx_hbm.at[0, pl.ds(i0, _IBLK)],
                              idxs[b], si[b]).wait()

    def fire_gather(b):
        pltpu.async_copy(tab_hbm.at[idxs[b]], rows[b], sg[b])

    def wait_gather(b):
        pltpu.make_async_copy(tab_hbm.at[idxs[b]], rows[b], sg[b]).wait()

    def fire_wb(c, t):
        for dt in range(8):
            pltpu.async_copy(trs[t].at[pl.ds(dt * 1024, 1024)],
                             out_hbm.at[c, dt, wid], sw[t])

    def wait_wb(t):
        for dt in range(8):
            pltpu.make_async_copy(trs[t].at[pl.ds(0, 1024)],
                                  out_hbm.at[0, 0, wid], sw[t]).wait()

    iota = jnp.arange(16, dtype=jnp.int32)
    rot = [(iota + s) % 16 for s in range(16)]
    ivec = [16 * g + iota for g in range(_IBLK // 16)]
    mvec = [((iota + s) % 16) * 128 + iota for s in range(16)]

    def transpose_chunk(b, t):
        @plsc.parallel_loop(0, 4, unroll=1)
        def _(dblk):
            d0 = dblk * 16
            d0v = jnp.full((16,), d0, dtype=jnp.int32)
            ng = _IBLK // 16
            prev = None
            for s in range(16):
                dv = d0v + rot[s]
                vs = [plsc.load_gather(rows[b], [ivec[g], dv])
                      for g in range(ng)]
                if prev is not None:
                    sp, vp = prev
                    for g in range(ng):
                        plsc.store_scatter(
                            trs[t], [mvec[sp] + (d0 * 128 + 16 * g)], vp[g])
                prev = (s, vs)
            sp, vp = prev
            for g in range(ng):
                plsc.store_scatter(
                    trs[t], [mvec[sp] + (d0 * 128 + 16 * g)], vp[g])

    for c in range(_A + 1):
        fire_idx(c, c % _NBUF)
    for c in range(_A):
        wait_idx(c % _NBUF)
        fire_gather(c % _NBUF)

    _STEP = 4

    def group(g, carry):
        for s in range(_STEP):
            k = g * _STEP + s
            b = s % _NBUF
            t = s % _NTB
            ba = (s + _A) % _NBUF
            bn = (s + _A + 1) % _NBUF

            @pl.when(k + _A < _NCHUNK)
            def _():
                wait_idx(ba)
                fire_gather(ba)

            wait_gather(b)

            @pl.when(k >= _NTB)
            def _():
                wait_wb(t)

            transpose_chunk(b, t)
            fire_wb(k, t)

            @pl.when(k + _A + 1 < _NCHUNK)
            def _():
                fire_idx(k + _A + 1, bn)
        return carry

    lax.fori_loop(0, _NCHUNK // _STEP, group, 0)

    for t in range(_NTB):
        wait_wb(t)


_mesh = plsc.VectorSubcoreMesh(core_axis_name="c", subcore_axis_name="s")


def kernel(tensor, table_fix, table_v):
    idx = jnp.swapaxes(tensor, 0, 1).astype(jnp.int32)
    tab = table_fix.at[_VOCAB - 3:].add(table_v)
    call = pl.kernel(
        _gather_body,
        out_type=jax.ShapeDtypeStruct((_J, 8, _NW, 1024), jnp.float32),
        mesh=_mesh,
        scratch_types=(
            [pltpu.VMEM((_IBLK,), jnp.int32) for _ in range(_NBUF)]
            + [pltpu.VMEM((_IBLK, _DIM), jnp.float32) for _ in range(_NBUF)]
            + [pltpu.VMEM((8 * 1024,), jnp.float32) for _ in range(_NTB)]
            + [pltpu.SemaphoreType.DMA for _ in range(2 * _NBUF + _NTB)]
        ),
        compiler_params=pltpu.CompilerParams(use_tc_tiling_on_sc=False,
                                             needs_layout_passes=False),
    )
    out4 = call(idx, tab)
    out5 = out4.reshape(_J, 8, _NW, 8, 128)
    out = jnp.transpose(out5, (0, 2, 4, 1, 3)).reshape(_J, _B, _DIM)
    return out
